# baseline (speedup 1.0000x reference)
import functools

import jax
import jax.numpy as jnp
from jax import lax
from jax.experimental import pallas as pl
from jax.experimental.pallas import tpu as pltpu
from jax.experimental.pallas import tpu_sc as plsc

N = 100000
E = 3200000
G = 128

NC = 2
NS = 16
H = N // NC
HP = 3136 * NS

KE = 1024
NCHUNK_ALL = E // KE
NCHUNK_TILE = NCHUNK_ALL // NS
NCHUNK_TAIL = NCHUNK_ALL - NS * NCHUNK_TILE
SUP = 16
NSUP = NCHUNK_TILE // SUP
REM = NCHUNK_TILE - NSUP * SUP

_MESH = dict(core_axis_name="c", subcore_axis_name="s", num_cores=NC,
             num_subcores=NS)


def _sc_deg_body(dst_h, ones_h, zeros_h, out, dstb, si0, si1, si2, si3,
                 ones_v, ss0, ss1, ss2, ss3, acc):
    c = lax.axis_index("c")
    s = lax.axis_index("s")
    lo = c * H
    sidxs = (si0, si1, si2, si3)
    ssems = (ss0, ss1, ss2, ss3)
    sidxb = si0

    pltpu.sync_copy(zeros_h, ones_v)
    for t in range(3):
        pltpu.sync_copy(ones_v, acc.at[pl.ds(s * 3136 + t * KE, KE)])
    pltpu.sync_copy(ones_v.at[pl.ds(0, 64)],
                    acc.at[pl.ds(s * 3136 + 3 * KE, 64)])
    pltpu.sync_copy(ones_h, ones_v)
    plsc.subcore_barrier()

    R = 4

    def do_superstep(base_chunk, n):
        pltpu.sync_copy(dst_h.at[pl.ds(base_chunk * KE, n * KE)],
                        dstb.at[pl.ds(0, n * KE)])
        sds = [None] * n
        for j in range(n):
            p = j % R
            if j - R >= 0:
                sds[j - R].wait()
            for i in range(KE // 16):
                d = dstb[pl.ds(j * KE + i * 16, 16)]
                ok = (d >= lo) & (d < lo + H)
                sidxs[p][pl.ds(i * 16, 16)] = jnp.where(ok, d - lo, H)
            sds[j] = pltpu.async_copy(ones_v, acc.at[sidxs[p]], ssems[p],
                                      add=True)
        for j in range(max(0, n - R), n):
            if sds[j] is not None:
                sds[j].wait()

    def body(ss, _):
        do_superstep(s * NCHUNK_TILE + ss * SUP, SUP)
        return 0

    lax.fori_loop(0, NSUP, body, 0)
    do_superstep(s * NCHUNK_TILE + NSUP * SUP, REM)

    @pl.when(s < NCHUNK_TAIL)
    def _():
        do_superstep(NS * NCHUNK_TILE + s, 1)

    plsc.subcore_barrier()

    def copy_out(src_off, dst_off, n):
        pltpu.sync_copy(acc.at[pl.ds(src_off, n)], ones_v.at[pl.ds(0, n)])
        pltpu.sync_copy(ones_v.at[pl.ds(0, n)], out.at[pl.ds(dst_off, n)])

    for t in range(3):
        copy_out(s * 3120 + t * KE, c * H + s * 3120 + t * KE, KE)
    copy_out(s * 3120 + 3 * KE, c * H + s * 3120 + 3 * KE, 3120 - 3 * KE)

    @pl.when(s == 0)
    def _():
        copy_out(NS * 3120, c * H + NS * 3120, H - NS * 3120)


def _sc_deg(dst_h, ones_h, zeros_h):
    return pl.kernel(
        _sc_deg_body,
        out_type=jax.ShapeDtypeStruct((N,), jnp.float32),
        mesh=plsc.VectorSubcoreMesh(**_MESH),
        compiler_params=pltpu.CompilerParams(use_tc_tiling_on_sc=False),
        scratch_types=(
            [pltpu.VMEM((SUP * KE,), jnp.int32)] +
            [pltpu.VMEM((KE,), jnp.int32)] * 4 +
            [pltpu.VMEM((KE,), jnp.float32)] +
            [pltpu.SemaphoreType.DMA] * 4 +
            [pltpu.VMEM_SHARED((HP,), jnp.float32)]
        ),
    )(dst_h, ones_h, zeros_h)


def _sc_agg_body(D, src_h, dst_h, table, zerosD, out, srcb, dstb,
                 si0, si1, si2, si3, r0, r1, r2, r3,
                 g0, g1, g2, g3, ss0, ss1, ss2, ss3, acc):
    c = lax.axis_index("c")
    s = lax.axis_index("s")
    lo = c * H
    rows = (r0, r1)
    sidxs = (si0, si1)
    gsems = (g0, g1)
    ssems = (ss0, ss1)
    rowb = r0

    pltpu.sync_copy(zerosD, rowb)
    for t in range(3):
        pltpu.sync_copy(rowb, acc.at[pl.ds(s * 3136 + t * KE, KE), :])
    pltpu.sync_copy(rowb.at[pl.ds(0, 64), :],
                    acc.at[pl.ds(s * 3136 + 3 * KE, 64), :])
    plsc.subcore_barrier()

    R = 2
    PF = 2

    def compute_sidx(j, sx):
        for i in range(KE // 16):
            d = dstb[pl.ds(j * KE + i * 16, 16)]
            ok = (d >= lo) & (d < lo + H)
            sx[pl.ds(i * 16, 16)] = jnp.where(ok, d - lo, H)

    def do_superstep(base_chunk, n):
        off = base_chunk * KE
        pltpu.sync_copy(src_h.at[pl.ds(off, n * KE)],
                        srcb.at[pl.ds(0, n * KE)])
        pltpu.sync_copy(dst_h.at[pl.ds(off, n * KE)],
                        dstb.at[pl.ds(0, n * KE)])

        def gather(j):
            return pltpu.async_copy(
                table.at[srcb.at[pl.ds(j * KE, KE)]], rows[j % R],
                gsems[j % R])

        gds = [None] * n
        sds = [None] * n
        for j in range(min(PF, n)):
            gds[j] = gather(j)
        for j in range(n):
            p = j % R
            gds[j].wait()
            compute_sidx(j, sidxs[p])
            sds[j] = pltpu.async_copy(rows[p], acc.at[sidxs[p]], ssems[p],
                                      add=True)
            if j + PF < n:
                if j + PF - R >= 0:
                    sds[j + PF - R].wait()
                gds[j + PF] = gather(j + PF)
        for j in range(max(0, n - R), n):
            if sds[j] is not None:
                sds[j].wait()

    def body(ss, _):
        do_superstep(s * NCHUNK_TILE + ss * SUP, SUP)
        return 0

    lax.fori_loop(0, NSUP, body, 0)
    do_superstep(s * NCHUNK_TILE + NSUP * SUP, REM)

    @pl.when(s < NCHUNK_TAIL)
    def _():
        do_superstep(NS * NCHUNK_TILE + s, 1)

    plsc.subcore_barrier()

    def copy_out(src_off, dst_off, n):
        pltpu.sync_copy(acc.at[pl.ds(src_off, n), :], rowb.at[pl.ds(0, n), :])
        pltpu.sync_copy(rowb.at[pl.ds(0, n), :], out.at[pl.ds(dst_off, n), :])

    for t in range(3):
        copy_out(s * 3120 + t * KE, c * H + s * 3120 + t * KE, KE)
    copy_out(s * 3120 + 3 * KE, c * H + s * 3120 + 3 * KE, 3120 - 3 * KE)

    @pl.when(s == 0)
    def _():
        copy_out(NS * 3120, c * H + NS * 3120, H - NS * 3120)


def _sc_agg(D, src_h, dst_h, table, zerosD):
    return pl.kernel(
        functools.partial(_sc_agg_body, D),
        out_type=jax.ShapeDtypeStruct((N, D), jnp.float32),
        mesh=plsc.VectorSubcoreMesh(**_MESH),
        compiler_params=pltpu.CompilerParams(use_tc_tiling_on_sc=False),
        scratch_types=(
            [pltpu.VMEM((SUP * KE,), jnp.int32)] * 2 +
            [pltpu.VMEM((KE,), jnp.int32)] * 4 +
            [pltpu.VMEM((KE, D), jnp.float32)] * 4 +
            [pltpu.SemaphoreType.DMA] * 8 +
            [pltpu.VMEM_SHARED((HP, D), jnp.float32)]
        ),
    )(src_h, dst_h, table, zerosD)


_POOL_PT = 3120
_POOL_CH = 1040


def _sc_maxpool_body(h1, h2, batch, out1, out2, bb, h1b, h2b, acc1, acc2,
                     sem):
    c = lax.axis_index("c")
    s = lax.axis_index("s")
    w = s * NC + c
    neg = jnp.full((16,), -jnp.inf, jnp.float32)

    def init(r, _):
        acc1[r, pl.ds(0, 16)] = neg
        acc1[r, pl.ds(16, 16)] = neg
        acc2[r, pl.ds(0, 16)] = neg
        acc2[r, pl.ds(16, 16)] = neg
        return 0

    lax.fori_loop(0, G, init, 0)

    def scan_range(base, count):
        pltpu.sync_copy(batch.at[pl.ds(base, count)], bb.at[pl.ds(0, count)])
        pltpu.sync_copy(h1.at[pl.ds(base, count), :],
                        h1b.at[pl.ds(0, count), :])
        pltpu.sync_copy(h2.at[pl.ds(base, count), :],
                        h2b.at[pl.ds(0, count), :])

        def body(i, _):
            b = bb[pl.ds(i, 16)][0]
            for accr, hb in ((acc1, h1b), (acc2, h2b)):
                for half in (0, 16):
                    cur = accr[b, pl.ds(half, 16)]
                    val = hb[i, pl.ds(half, 16)]
                    accr[b, pl.ds(half, 16)] = jnp.maximum(cur, val)
            return 0

        lax.fori_loop(0, count, body, 0)

    for k in range(_POOL_PT // _POOL_CH):
        scan_range(w * _POOL_PT + k * _POOL_CH, _POOL_CH)

    @pl.when(w == 0)
    def _():
        scan_range(NC * NS * _POOL_PT, N - NC * NS * _POOL_PT)

    pltpu.sync_copy(acc1, out1.at[w])
    pltpu.sync_copy(acc2, out2.at[w])


def _sc_maxpool(h1, h2, batch):
    return pl.kernel(
        _sc_maxpool_body,
        out_type=[jax.ShapeDtypeStruct((NC * NS, G, 32), jnp.float32),
                  jax.ShapeDtypeStruct((NC * NS, G, 32), jnp.float32)],
        mesh=plsc.VectorSubcoreMesh(**_MESH),
        compiler_params=pltpu.CompilerParams(use_tc_tiling_on_sc=False),
        scratch_types=[
            pltpu.VMEM((_POOL_CH + 16,), jnp.int32),
            pltpu.VMEM((_POOL_CH, 32), jnp.float32),
            pltpu.VMEM((_POOL_CH, 32), jnp.float32),
            pltpu.VMEM((G, 32), jnp.float32),
            pltpu.VMEM((G, 32), jnp.float32),
            pltpu.SemaphoreType.DMA,
        ],
    )(h1, h2, batch)


_BLK = 2000
_NBLK = N // _BLK


def _tc_prep(deg, x):
    def body(deg_ref, x_ref, dinv_ref, u_ref):
        dinv = lax.rsqrt(deg_ref[...] + 1.0)
        dinv_ref[...] = dinv
        xb = x_ref[...]
        u_ref[...] = jnp.concatenate(
            [dinv * xb, jnp.zeros((_BLK, 13), jnp.float32)], axis=1)

    return pl.pallas_call(
        body,
        grid=(_NBLK,),
        in_specs=[pl.BlockSpec((_BLK, 1), lambda i: (i, 0)),
                  pl.BlockSpec((_BLK, 3), lambda i: (i, 0))],
        out_specs=[pl.BlockSpec((_BLK, 1), lambda i: (i, 0)),
                   pl.BlockSpec((_BLK, 16), lambda i: (i, 0))],
        out_shape=[jax.ShapeDtypeStruct((N, 1), jnp.float32),
                   jax.ShapeDtypeStruct((N, 16), jnp.float32)],
    )(deg, x)


def _tc_layer1(agg1, u, dinv, W1p, b1):
    def body(agg_ref, u_ref, dinv_ref, w_ref, b_ref, h1_ref, y1a_ref,
             y1b_ref):
        dinv = dinv_ref[...]
        pre = dinv * (agg_ref[...] + u_ref[...])
        h1 = jnp.maximum(
            jnp.dot(pre, w_ref[...],
                    preferred_element_type=jnp.float32) + b_ref[...], 0.0)
        h1_ref[...] = h1
        y1 = dinv * h1
        y1a_ref[...] = y1[:, :16]
        y1b_ref[...] = y1[:, 16:]

    return pl.pallas_call(
        body,
        grid=(_NBLK,),
        in_specs=[pl.BlockSpec((_BLK, 16), lambda i: (i, 0)),
                  pl.BlockSpec((_BLK, 16), lambda i: (i, 0)),
                  pl.BlockSpec((_BLK, 1), lambda i: (i, 0)),
                  pl.BlockSpec((16, 32), lambda i: (0, 0)),
                  pl.BlockSpec((1, 32), lambda i: (0, 0))],
        out_specs=[pl.BlockSpec((_BLK, 32), lambda i: (i, 0)),
                   pl.BlockSpec((_BLK, 16), lambda i: (i, 0)),
                   pl.BlockSpec((_BLK, 16), lambda i: (i, 0))],
        out_shape=[jax.ShapeDtypeStruct((N, 32), jnp.float32),
                   jax.ShapeDtypeStruct((N, 16), jnp.float32),
                   jax.ShapeDtypeStruct((N, 16), jnp.float32)],
    )(agg1, u, dinv, W1p, b1)


def _tc_layer2(agg2a, agg2b, h1, dinv, batch2d, W2, b2):
    def body(agga_ref, aggb_ref, h1_ref, dinv_ref, batch_ref, w_ref, b_ref,
             h2_ref, sum1_ref, sum2_ref, cnt_ref):
        i = pl.program_id(0)
        dinv = dinv_ref[...]
        h1 = h1_ref[...]
        agg = jnp.concatenate([agga_ref[...], aggb_ref[...]], axis=1)
        pre = dinv * agg + (dinv * dinv) * h1
        h2 = jnp.maximum(
            jnp.dot(pre, w_ref[...],
                    preferred_element_type=jnp.float32) + b_ref[...], 0.0)
        h2_ref[...] = h2
        gids = lax.broadcasted_iota(jnp.int32, (_BLK, G), 1)
        oh = jnp.where(batch_ref[...] == gids, 1.0, 0.0).astype(jnp.float32)
        dn = (((0,), (0,)), ((), ()))
        s1 = lax.dot_general(oh, h1, dn, preferred_element_type=jnp.float32)
        s2 = lax.dot_general(oh, h2, dn, preferred_element_type=jnp.float32)
        ct = lax.dot_general(oh, jnp.ones((_BLK, 1), jnp.float32), dn,
                             preferred_element_type=jnp.float32)

        @pl.when(i == 0)
        def _():
            sum1_ref[...] = jnp.zeros_like(sum1_ref)
            sum2_ref[...] = jnp.zeros_like(sum2_ref)
            cnt_ref[...] = jnp.zeros_like(cnt_ref)

        sum1_ref[...] += s1
        sum2_ref[...] += s2
        cnt_ref[...] += ct

    return pl.pallas_call(
        body,
        grid=(_NBLK,),
        in_specs=[pl.BlockSpec((_BLK, 16), lambda i: (i, 0)),
                  pl.BlockSpec((_BLK, 16), lambda i: (i, 0)),
                  pl.BlockSpec((_BLK, 32), lambda i: (i, 0)),
                  pl.BlockSpec((_BLK, 1), lambda i: (i, 0)),
                  pl.BlockSpec((_BLK, 1), lambda i: (i, 0)),
                  pl.BlockSpec((32, 32), lambda i: (0, 0)),
                  pl.BlockSpec((1, 32), lambda i: (0, 0))],
        out_specs=[pl.BlockSpec((_BLK, 32), lambda i: (i, 0)),
                   pl.BlockSpec((G, 32), lambda i: (0, 0)),
                   pl.BlockSpec((G, 32), lambda i: (0, 0)),
                   pl.BlockSpec((G, 1), lambda i: (0, 0))],
        out_shape=[jax.ShapeDtypeStruct((N, 32), jnp.float32),
                   jax.ShapeDtypeStruct((G, 32), jnp.float32),
                   jax.ShapeDtypeStruct((G, 32), jnp.float32),
                   jax.ShapeDtypeStruct((G, 1), jnp.float32)],
    )(agg2a, agg2b, h1, dinv, batch2d, W2, b2)


def _tc_head(mp1, mp2, sum1, sum2, cnt, LW1, Lb1, LW2, Lb2):
    def body(mp1_ref, mp2_ref, s1_ref, s2_ref, cnt_ref, lw1_ref, lb1_ref,
             lw2_ref, lb2_ref, out_ref):
        max1 = mp1_ref[0]
        max2 = mp2_ref[0]
        for t in range(1, NC * NS):
            max1 = jnp.maximum(max1, mp1_ref[t])
            max2 = jnp.maximum(max2, mp2_ref[t])
        invc = 1.0 / jnp.maximum(cnt_ref[...], 1.0)
        x1 = jnp.concatenate([max1, s1_ref[...] * invc], axis=1)
        x2 = jnp.concatenate([max2, s2_ref[...] * invc], axis=1)
        z = jnp.maximum(
            jnp.dot(x1 + x2, lw1_ref[...],
                    preferred_element_type=jnp.float32) + lb1_ref[...], 0.0)
        out_ref[...] = jnp.dot(
            z, lw2_ref[...], preferred_element_type=jnp.float32) + lb2_ref[...]

    return pl.pallas_call(
        body,
        out_shape=jax.ShapeDtypeStruct((G, 64), jnp.float32),
    )(mp1, mp2, sum1, sum2, cnt, LW1, Lb1, LW2, Lb2)


def kernel(x, edge_index, batch, W1, b1, W2, b2, LW1, Lb1, LW2, Lb2):
    src = edge_index[0]
    dst = edge_index[1]
    ones1 = jnp.ones((KE,), jnp.float32)
    zeros1 = jnp.zeros((KE,), jnp.float32)
    zeros16 = jnp.zeros((KE, 16), jnp.float32)
    W1p = jnp.concatenate([W1, jnp.zeros((13, 32), jnp.float32)], axis=0)

    deg = _sc_deg(dst, ones1, zeros1)
    dinv, u = _tc_prep(deg.reshape(N, 1), x)
    agg1 = _sc_agg(16, src, dst, u, zeros16)
    h1, y1a, y1b = _tc_layer1(agg1, u, dinv, W1p, b1.reshape(1, 32))
    agg2a = _sc_agg(16, src, dst, y1a, zeros16)
    agg2b = _sc_agg(16, src, dst, y1b, zeros16)
    h2, sum1, sum2, cnt = _tc_layer2(agg2a, agg2b, h1, dinv,
                                     batch.reshape(N, 1),
                                     W2, b2.reshape(1, 32))
    mp1, mp2 = _sc_maxpool(h1, h2, batch)
    return _tc_head(mp1, mp2, sum1, sum2, cnt, LW1, Lb1.reshape(1, 64),
                    LW2, Lb2.reshape(1, 64))


# speedup vs baseline: 4.5498x; 4.5498x over previous
import functools

import jax
import jax.numpy as jnp
from jax import lax
from jax.experimental import pallas as pl
from jax.experimental.pallas import tpu as pltpu
from jax.experimental.pallas import tpu_sc as plsc

N = 100000
E = 3200000
G = 128

NC = 2
NS = 16
H = N // NC
HP = 3136 * NS

KE = 1024
NCHUNK_ALL = E // KE
NCHUNK_TILE = NCHUNK_ALL // NS
NCHUNK_TAIL = NCHUNK_ALL - NS * NCHUNK_TILE
SUP = 16
NSUP = NCHUNK_TILE // SUP
REM = NCHUNK_TILE - NSUP * SUP

_MESH = dict(core_axis_name="c", subcore_axis_name="s", num_cores=NC,
             num_subcores=NS)


def _sc_deg_body(dst_h, ones_h, zeros_h, out, dstb, si0, si1, si2, si3,
                 ones_v, ss0, ss1, ss2, ss3, acc):
    c = lax.axis_index("c")
    s = lax.axis_index("s")
    lo = c * H
    sidxs = (si0, si1, si2, si3)
    ssems = (ss0, ss1, ss2, ss3)
    sidxb = si0

    pltpu.sync_copy(zeros_h, ones_v)
    for t in range(3):
        pltpu.sync_copy(ones_v, acc.at[pl.ds(s * 3136 + t * KE, KE)])
    pltpu.sync_copy(ones_v.at[pl.ds(0, 64)],
                    acc.at[pl.ds(s * 3136 + 3 * KE, 64)])
    pltpu.sync_copy(ones_h, ones_v)
    plsc.subcore_barrier()

    R = 4

    def do_superstep(base_chunk, n):
        pltpu.sync_copy(dst_h.at[pl.ds(base_chunk * KE, n * KE)],
                        dstb.at[pl.ds(0, n * KE)])
        sds = [None] * n
        for j in range(n):
            p = j % R
            if j - R >= 0:
                sds[j - R].wait()
            iota = lax.iota(jnp.int32, 16)
            for i in range(KE // 16):
                d = dstb[pl.ds(j * KE + i * 16, 16)]
                ok = (d >= lo) & (d < lo + H)
                trash = (H + (i % 8) * 16) + iota
                sidxs[p][pl.ds(i * 16, 16)] = jnp.where(ok, d - lo, trash)
            sds[j] = pltpu.async_copy(ones_v, acc.at[sidxs[p]], ssems[p],
                                      add=True)
        for j in range(max(0, n - R), n):
            if sds[j] is not None:
                sds[j].wait()

    def body(ss, _):
        do_superstep(s * NCHUNK_TILE + ss * SUP, SUP)
        return 0

    lax.fori_loop(0, NSUP, body, 0)
    do_superstep(s * NCHUNK_TILE + NSUP * SUP, REM)

    @pl.when(s < NCHUNK_TAIL)
    def _():
        do_superstep(NS * NCHUNK_TILE + s, 1)

    plsc.subcore_barrier()

    def copy_out(src_off, dst_off, n):
        pltpu.sync_copy(acc.at[pl.ds(src_off, n)], ones_v.at[pl.ds(0, n)])
        pltpu.sync_copy(ones_v.at[pl.ds(0, n)], out.at[pl.ds(dst_off, n)])

    for t in range(3):
        copy_out(s * 3120 + t * KE, c * H + s * 3120 + t * KE, KE)
    copy_out(s * 3120 + 3 * KE, c * H + s * 3120 + 3 * KE, 3120 - 3 * KE)

    @pl.when(s == 0)
    def _():
        copy_out(NS * 3120, c * H + NS * 3120, H - NS * 3120)


def _sc_deg(dst_h, ones_h, zeros_h):
    return pl.kernel(
        _sc_deg_body,
        out_type=jax.ShapeDtypeStruct((N,), jnp.float32),
        mesh=plsc.VectorSubcoreMesh(**_MESH),
        compiler_params=pltpu.CompilerParams(use_tc_tiling_on_sc=False),
        scratch_types=(
            [pltpu.VMEM((SUP * KE,), jnp.int32)] +
            [pltpu.VMEM((KE,), jnp.int32)] * 4 +
            [pltpu.VMEM((KE,), jnp.float32)] +
            [pltpu.SemaphoreType.DMA] * 4 +
            [pltpu.VMEM_SHARED((HP,), jnp.float32)]
        ),
    )(dst_h, ones_h, zeros_h)


def _sc_agg_body(D, src_h, dst_h, table, zerosD, out, srcb, dstb,
                 si0, si1, si2, si3, r0, r1, r2, r3,
                 g0, g1, g2, g3, ss0, ss1, ss2, ss3, acc):
    c = lax.axis_index("c")
    s = lax.axis_index("s")
    lo = c * H
    rows = (r0, r1)
    sidxs = (si0, si1)
    gsems = (g0, g1)
    ssems = (ss0, ss1)
    rowb = r0

    pltpu.sync_copy(zerosD, rowb)
    for t in range(3):
        pltpu.sync_copy(rowb, acc.at[pl.ds(s * 3136 + t * KE, KE), :])
    pltpu.sync_copy(rowb.at[pl.ds(0, 64), :],
                    acc.at[pl.ds(s * 3136 + 3 * KE, 64), :])
    plsc.subcore_barrier()

    R = 2
    PF = 2

    def compute_sidx(j, sx):
        iota = lax.iota(jnp.int32, 16)
        for i in range(KE // 16):
            d = dstb[pl.ds(j * KE + i * 16, 16)]
            ok = (d >= lo) & (d < lo + H)
            trash = (H + (i % 8) * 16) + iota
            sx[pl.ds(i * 16, 16)] = jnp.where(ok, d - lo, trash)

    def do_superstep(base_chunk, n):
        off = base_chunk * KE
        pltpu.sync_copy(src_h.at[pl.ds(off, n * KE)],
                        srcb.at[pl.ds(0, n * KE)])
        pltpu.sync_copy(dst_h.at[pl.ds(off, n * KE)],
                        dstb.at[pl.ds(0, n * KE)])

        def gather(j):
            return pltpu.async_copy(
                table.at[srcb.at[pl.ds(j * KE, KE)]], rows[j % R],
                gsems[j % R])

        gds = [None] * n
        sds = [None] * n
        for j in range(min(PF, n)):
            gds[j] = gather(j)
        for j in range(n):
            p = j % R
            gds[j].wait()
            compute_sidx(j, sidxs[p])
            sds[j] = pltpu.async_copy(rows[p], acc.at[sidxs[p]], ssems[p],
                                      add=True)
            if j + PF < n:
                if j + PF - R >= 0:
                    sds[j + PF - R].wait()
                gds[j + PF] = gather(j + PF)
        for j in range(max(0, n - R), n):
            if sds[j] is not None:
                sds[j].wait()

    def body(ss, _):
        do_superstep(s * NCHUNK_TILE + ss * SUP, SUP)
        return 0

    lax.fori_loop(0, NSUP, body, 0)
    do_superstep(s * NCHUNK_TILE + NSUP * SUP, REM)

    @pl.when(s < NCHUNK_TAIL)
    def _():
        do_superstep(NS * NCHUNK_TILE + s, 1)

    plsc.subcore_barrier()

    def copy_out(src_off, dst_off, n):
        pltpu.sync_copy(acc.at[pl.ds(src_off, n), :], rowb.at[pl.ds(0, n), :])
        pltpu.sync_copy(rowb.at[pl.ds(0, n), :], out.at[pl.ds(dst_off, n), :])

    for t in range(3):
        copy_out(s * 3120 + t * KE, c * H + s * 3120 + t * KE, KE)
    copy_out(s * 3120 + 3 * KE, c * H + s * 3120 + 3 * KE, 3120 - 3 * KE)

    @pl.when(s == 0)
    def _():
        copy_out(NS * 3120, c * H + NS * 3120, H - NS * 3120)


def _sc_agg(D, src_h, dst_h, table, zerosD):
    return pl.kernel(
        functools.partial(_sc_agg_body, D),
        out_type=jax.ShapeDtypeStruct((N, D), jnp.float32),
        mesh=plsc.VectorSubcoreMesh(**_MESH),
        compiler_params=pltpu.CompilerParams(use_tc_tiling_on_sc=False),
        scratch_types=(
            [pltpu.VMEM((SUP * KE,), jnp.int32)] * 2 +
            [pltpu.VMEM((KE,), jnp.int32)] * 4 +
            [pltpu.VMEM((KE, D), jnp.float32)] * 4 +
            [pltpu.SemaphoreType.DMA] * 8 +
            [pltpu.VMEM_SHARED((HP, D), jnp.float32)]
        ),
    )(src_h, dst_h, table, zerosD)


_POOL_PT = 3120
_POOL_CH = 1040


def _sc_maxpool_body(h1, h2, batch, out1, out2, bb, h1b, h2b, acc1, acc2,
                     sem):
    c = lax.axis_index("c")
    s = lax.axis_index("s")
    w = s * NC + c
    neg = jnp.full((16,), -jnp.inf, jnp.float32)

    def init(r, _):
        acc1[r, pl.ds(0, 16)] = neg
        acc1[r, pl.ds(16, 16)] = neg
        acc2[r, pl.ds(0, 16)] = neg
        acc2[r, pl.ds(16, 16)] = neg
        return 0

    lax.fori_loop(0, G, init, 0)

    def scan_range(base, count):
        pltpu.sync_copy(batch.at[pl.ds(base, count)], bb.at[pl.ds(0, count)])
        pltpu.sync_copy(h1.at[pl.ds(base, count), :],
                        h1b.at[pl.ds(0, count), :])
        pltpu.sync_copy(h2.at[pl.ds(base, count), :],
                        h2b.at[pl.ds(0, count), :])

        def body(i, _):
            b = bb[pl.ds(i, 16)][0]
            for accr, hb in ((acc1, h1b), (acc2, h2b)):
                for half in (0, 16):
                    cur = accr[b, pl.ds(half, 16)]
                    val = hb[i, pl.ds(half, 16)]
                    accr[b, pl.ds(half, 16)] = jnp.maximum(cur, val)
            return 0

        lax.fori_loop(0, count, body, 0)

    for k in range(_POOL_PT // _POOL_CH):
        scan_range(w * _POOL_PT + k * _POOL_CH, _POOL_CH)

    @pl.when(w == 0)
    def _():
        scan_range(NC * NS * _POOL_PT, N - NC * NS * _POOL_PT)

    pltpu.sync_copy(acc1, out1.at[w])
    pltpu.sync_copy(acc2, out2.at[w])


def _sc_maxpool(h1, h2, batch):
    return pl.kernel(
        _sc_maxpool_body,
        out_type=[jax.ShapeDtypeStruct((NC * NS, G, 32), jnp.float32),
                  jax.ShapeDtypeStruct((NC * NS, G, 32), jnp.float32)],
        mesh=plsc.VectorSubcoreMesh(**_MESH),
        compiler_params=pltpu.CompilerParams(use_tc_tiling_on_sc=False),
        scratch_types=[
            pltpu.VMEM((_POOL_CH + 16,), jnp.int32),
            pltpu.VMEM((_POOL_CH, 32), jnp.float32),
            pltpu.VMEM((_POOL_CH, 32), jnp.float32),
            pltpu.VMEM((G, 32), jnp.float32),
            pltpu.VMEM((G, 32), jnp.float32),
            pltpu.SemaphoreType.DMA,
        ],
    )(h1, h2, batch)


_BLK = 2000
_NBLK = N // _BLK


def _tc_prep(deg, x):
    def body(deg_ref, x_ref, dinv_ref, u_ref):
        dinv = lax.rsqrt(deg_ref[...] + 1.0)
        dinv_ref[...] = dinv
        xb = x_ref[...]
        u_ref[...] = jnp.concatenate(
            [dinv * xb, jnp.zeros((_BLK, 13), jnp.float32)], axis=1)

    return pl.pallas_call(
        body,
        grid=(_NBLK,),
        in_specs=[pl.BlockSpec((_BLK, 1), lambda i: (i, 0)),
                  pl.BlockSpec((_BLK, 3), lambda i: (i, 0))],
        out_specs=[pl.BlockSpec((_BLK, 1), lambda i: (i, 0)),
                   pl.BlockSpec((_BLK, 16), lambda i: (i, 0))],
        out_shape=[jax.ShapeDtypeStruct((N, 1), jnp.float32),
                   jax.ShapeDtypeStruct((N, 16), jnp.float32)],
    )(deg, x)


def _tc_layer1(agg1, u, dinv, W1p, b1):
    def body(agg_ref, u_ref, dinv_ref, w_ref, b_ref, h1_ref, y1a_ref,
             y1b_ref):
        dinv = dinv_ref[...]
        pre = dinv * (agg_ref[...] + u_ref[...])
        h1 = jnp.maximum(
            jnp.dot(pre, w_ref[...],
                    preferred_element_type=jnp.float32) + b_ref[...], 0.0)
        h1_ref[...] = h1
        y1 = dinv * h1
        y1a_ref[...] = y1[:, :16]
        y1b_ref[...] = y1[:, 16:]

    return pl.pallas_call(
        body,
        grid=(_NBLK,),
        in_specs=[pl.BlockSpec((_BLK, 16), lambda i: (i, 0)),
                  pl.BlockSpec((_BLK, 16), lambda i: (i, 0)),
                  pl.BlockSpec((_BLK, 1), lambda i: (i, 0)),
                  pl.BlockSpec((16, 32), lambda i: (0, 0)),
                  pl.BlockSpec((1, 32), lambda i: (0, 0))],
        out_specs=[pl.BlockSpec((_BLK, 32), lambda i: (i, 0)),
                   pl.BlockSpec((_BLK, 16), lambda i: (i, 0)),
                   pl.BlockSpec((_BLK, 16), lambda i: (i, 0))],
        out_shape=[jax.ShapeDtypeStruct((N, 32), jnp.float32),
                   jax.ShapeDtypeStruct((N, 16), jnp.float32),
                   jax.ShapeDtypeStruct((N, 16), jnp.float32)],
    )(agg1, u, dinv, W1p, b1)


def _tc_layer2(agg2a, agg2b, h1, dinv, batch2d, W2, b2):
    def body(agga_ref, aggb_ref, h1_ref, dinv_ref, batch_ref, w_ref, b_ref,
             h2_ref, sum1_ref, sum2_ref, cnt_ref):
        i = pl.program_id(0)
        dinv = dinv_ref[...]
        h1 = h1_ref[...]
        agg = jnp.concatenate([agga_ref[...], aggb_ref[...]], axis=1)
        pre = dinv * agg + (dinv * dinv) * h1
        h2 = jnp.maximum(
            jnp.dot(pre, w_ref[...],
                    preferred_element_type=jnp.float32) + b_ref[...], 0.0)
        h2_ref[...] = h2
        gids = lax.broadcasted_iota(jnp.int32, (_BLK, G), 1)
        oh = jnp.where(batch_ref[...] == gids, 1.0, 0.0).astype(jnp.float32)
        dn = (((0,), (0,)), ((), ()))
        s1 = lax.dot_general(oh, h1, dn, preferred_element_type=jnp.float32)
        s2 = lax.dot_general(oh, h2, dn, preferred_element_type=jnp.float32)
        ct = lax.dot_general(oh, jnp.ones((_BLK, 1), jnp.float32), dn,
                             preferred_element_type=jnp.float32)

        @pl.when(i == 0)
        def _():
            sum1_ref[...] = jnp.zeros_like(sum1_ref)
            sum2_ref[...] = jnp.zeros_like(sum2_ref)
            cnt_ref[...] = jnp.zeros_like(cnt_ref)

        sum1_ref[...] += s1
        sum2_ref[...] += s2
        cnt_ref[...] += ct

    return pl.pallas_call(
        body,
        grid=(_NBLK,),
        in_specs=[pl.BlockSpec((_BLK, 16), lambda i: (i, 0)),
                  pl.BlockSpec((_BLK, 16), lambda i: (i, 0)),
                  pl.BlockSpec((_BLK, 32), lambda i: (i, 0)),
                  pl.BlockSpec((_BLK, 1), lambda i: (i, 0)),
                  pl.BlockSpec((_BLK, 1), lambda i: (i, 0)),
                  pl.BlockSpec((32, 32), lambda i: (0, 0)),
                  pl.BlockSpec((1, 32), lambda i: (0, 0))],
        out_specs=[pl.BlockSpec((_BLK, 32), lambda i: (i, 0)),
                   pl.BlockSpec((G, 32), lambda i: (0, 0)),
                   pl.BlockSpec((G, 32), lambda i: (0, 0)),
                   pl.BlockSpec((G, 1), lambda i: (0, 0))],
        out_shape=[jax.ShapeDtypeStruct((N, 32), jnp.float32),
                   jax.ShapeDtypeStruct((G, 32), jnp.float32),
                   jax.ShapeDtypeStruct((G, 32), jnp.float32),
                   jax.ShapeDtypeStruct((G, 1), jnp.float32)],
    )(agg2a, agg2b, h1, dinv, batch2d, W2, b2)


def _tc_head(mp1, mp2, sum1, sum2, cnt, LW1, Lb1, LW2, Lb2):
    def body(mp1_ref, mp2_ref, s1_ref, s2_ref, cnt_ref, lw1_ref, lb1_ref,
             lw2_ref, lb2_ref, out_ref):
        max1 = mp1_ref[0]
        max2 = mp2_ref[0]
        for t in range(1, NC * NS):
            max1 = jnp.maximum(max1, mp1_ref[t])
            max2 = jnp.maximum(max2, mp2_ref[t])
        invc = 1.0 / jnp.maximum(cnt_ref[...], 1.0)
        x1 = jnp.concatenate([max1, s1_ref[...] * invc], axis=1)
        x2 = jnp.concatenate([max2, s2_ref[...] * invc], axis=1)
        z = jnp.maximum(
            jnp.dot(x1 + x2, lw1_ref[...],
                    preferred_element_type=jnp.float32) + lb1_ref[...], 0.0)
        out_ref[...] = jnp.dot(
            z, lw2_ref[...], preferred_element_type=jnp.float32) + lb2_ref[...]

    return pl.pallas_call(
        body,
        out_shape=jax.ShapeDtypeStruct((G, 64), jnp.float32),
    )(mp1, mp2, sum1, sum2, cnt, LW1, Lb1, LW2, Lb2)


def kernel(x, edge_index, batch, W1, b1, W2, b2, LW1, Lb1, LW2, Lb2):
    src = edge_index[0]
    dst = edge_index[1]
    ones1 = jnp.ones((KE,), jnp.float32)
    zeros1 = jnp.zeros((KE,), jnp.float32)
    zeros16 = jnp.zeros((KE, 16), jnp.float32)
    W1p = jnp.concatenate([W1, jnp.zeros((13, 32), jnp.float32)], axis=0)

    deg = _sc_deg(dst, ones1, zeros1)
    dinv, u = _tc_prep(deg.reshape(N, 1), x)
    agg1 = _sc_agg(16, src, dst, u, zeros16)
    h1, y1a, y1b = _tc_layer1(agg1, u, dinv, W1p, b1.reshape(1, 32))
    agg2a = _sc_agg(16, src, dst, y1a, zeros16)
    agg2b = _sc_agg(16, src, dst, y1b, zeros16)
    h2, sum1, sum2, cnt = _tc_layer2(agg2a, agg2b, h1, dinv,
                                     batch.reshape(N, 1),
                                     W2, b2.reshape(1, 32))
    mp1, mp2 = _sc_maxpool(h1, h2, batch)
    return _tc_head(mp1, mp2, sum1, sum2, cnt, LW1, Lb1.reshape(1, 64),
                    LW2, Lb2.reshape(1, 64))
